# Initial kernel scaffold; baseline (speedup 1.0000x reference)
import jax
import jax.numpy as jnp
from jax.experimental import pallas as pl


def kernel(atom_fea, nbr_fea, state_fea, atom_nbr_idx, node_atom_idx, W_atom, b_atom, W_embed, b_embed, W_fc, b_fc):
    raise NotImplementedError("write your pallas kernel here")


# speedup vs baseline: 6.0387x; 6.0387x over previous
import functools
import numpy as np
import jax
import jax.numpy as jnp
from jax import lax
from jax.experimental import pallas as pl
from jax.experimental.pallas import tpu as pltpu
from jax.experimental.pallas import tpu_sc as plsc

M = 16
A = 16
NB = 4
NGP = 512
BN = 2000
DMA_ROWS = 80
KJ = 5


def _softplus(x):
    return jnp.maximum(x, 0.0) + jnp.log(1.0 + jnp.exp(-jnp.abs(x)))


def _sigmoid(x):
    return 1.0 / (1.0 + jnp.exp(-x))


def _sc_gather(table, idx_flat):
    B = idx_flat.shape[0]
    info = plsc.get_sparse_core_info()
    nc, ns = info.num_cores, info.num_subcores
    nw = nc * ns
    chunk = KJ * DMA_ROWS
    n_outer = B // (nw * chunk)

    mesh = plsc.VectorSubcoreMesh(core_axis_name="c", subcore_axis_name="s")

    @functools.partial(
        pl.kernel,
        mesh=mesh,
        compiler_params=pltpu.CompilerParams(use_tc_tiling_on_sc=False),
        out_type=jax.ShapeDtypeStruct((B, A), jnp.float32),
        scratch_types=[
            pltpu.VMEM((chunk,), jnp.int32),
            pltpu.VMEM((chunk, A), jnp.float32),
            pltpu.SemaphoreType.DMA,
        ],
    )
    def gather_kernel(table_hbm, idx_hbm, out_hbm, idx_v, rows_v, sem):
        wid = lax.axis_index("s") * nc + lax.axis_index("c")

        def body(i, carry):
            g = wid * n_outer + i
            pltpu.sync_copy(idx_hbm.at[pl.ds(g * chunk, chunk)], idx_v)
            copies = [
                pltpu.async_copy(
                    table_hbm.at[idx_v.at[pl.ds(j * DMA_ROWS, DMA_ROWS)]],
                    rows_v.at[pl.ds(j * DMA_ROWS, DMA_ROWS)],
                    sem,
                )
                for j in range(KJ)
            ]
            for c in copies:
                c.wait()
            pltpu.sync_copy(rows_v, out_hbm.at[pl.ds(g * chunk, chunk)])
            return carry

        lax.fori_loop(0, n_outer, body, 0)

    return gather_kernel(table, idx_flat)


def _atom_body(af_ref, g1_ref, nb_ref, ni_ref, sp_ref, wself_ref, wst_ref,
               ba_ref, bdn_ref, bde_ref, t1m_ref, s16_ref, s4_ref, ssm_ref,
               xsm_ref, out_ref, ex_ref):
    af = af_ref[...]
    g1 = g1_ref[...]
    nb = nb_ref[...]
    ni = ni_ref[...]
    oh = (lax.broadcasted_iota(jnp.int32, (BN, NGP), 1) == ni).astype(jnp.float32)
    st = jnp.dot(oh, sp_ref[...], preferred_element_type=jnp.float32)
    t1 = (jnp.dot(af, wself_ref[...], preferred_element_type=jnp.float32)
          + jnp.dot(st, wst_ref[...], preferred_element_type=jnp.float32)
          + ba_ref[...])
    fc = (jnp.dot(g1, bdn_ref[...], preferred_element_type=jnp.float32)
          + jnp.dot(nb, bde_ref[...], preferred_element_type=jnp.float32)
          + jnp.dot(t1, t1m_ref[...], preferred_element_type=jnp.float32))
    filt = fc[:, :M * A]
    core = fc[:, M * A:]
    h = _sigmoid(filt) * _softplus(core)
    nbr_sumed = jnp.dot(h, s16_ref[...], preferred_element_type=jnp.float32)
    out_ref[...] = _softplus(af + nbr_sumed)
    sm = jnp.dot(nb, ssm_ref[...], preferred_element_type=jnp.float32)
    sn = jnp.dot(nb, s4_ref[...], preferred_element_type=jnp.float32)
    smexp = jnp.dot(sm, xsm_ref[...], preferred_element_type=jnp.float32)
    sns = jnp.dot(smexp * nb, s4_ref[...], preferred_element_type=jnp.float32)
    ex_ref[...] = jnp.concatenate([sn, sns, sm], axis=1)


def _sym_body(g2_ref, ex_ref, ni_ref, s16_ref, x16_ref, acc_ref):
    g2 = g2_ref[...]
    ex = ex_ref[...]
    sn = ex[:, 0:NB]
    sns = ex[:, NB:2 * NB]
    sm = ex[:, 2 * NB:2 * NB + M]
    gsum = jnp.dot(g2, s16_ref[...], preferred_element_type=jnp.float32)
    smexp = jnp.dot(sm, x16_ref[...], preferred_element_type=jnp.float32)
    gwsum = jnp.dot(smexp * g2, s16_ref[...], preferred_element_type=jnp.float32)
    a1 = (jnp.sum(sns * sns, axis=1, keepdims=True)
          + jnp.sum(gwsum * gwsum, axis=1, keepdims=True))
    b1 = (jnp.sum(sns * sn, axis=1, keepdims=True)
          + jnp.sum(gwsum * gsum, axis=1, keepdims=True))
    c1 = (jnp.sum(sn * sn, axis=1, keepdims=True)
          + jnp.sum(gsum * gsum, axis=1, keepdims=True))
    s1 = jnp.sum(sm, axis=1, keepdims=True)
    ones = jnp.ones_like(s1)
    zero = jnp.zeros_like(s1)
    vals = jnp.concatenate([a1, b1, c1, s1, s1 * s1, ones, zero, zero], axis=1)
    ni = ni_ref[...]
    oh = (lax.broadcasted_iota(jnp.int32, (BN, NGP), 1) == ni).astype(jnp.float32)
    part = lax.dot_general(oh, vals, (((0,), (0,)), ((), ())),
                           preferred_element_type=jnp.float32)

    @pl.when(pl.program_id(0) == 0)
    def _():
        acc_ref[...] = part

    @pl.when(pl.program_id(0) != 0)
    def _():
        acc_ref[...] = acc_ref[...] + part


def _final_body(acc_ref, sp_ref, r_ref, bfc_ref, out_ref):
    acc = acc_ref[...]
    cnt = jnp.maximum(acc[:, 5:6], 1.0)
    a1 = acc[:, 0:1] / cnt
    b1 = acc[:, 1:2] / cnt
    c1 = acc[:, 2:3] / cnt
    s1 = acc[:, 3:4] / cnt
    s2 = acc[:, 4:5] / cnt
    sp = sp_ref[...]
    q = jnp.sum(sp * sp, axis=1, keepdims=True)
    al = a1 + q * s2
    be = b1 + float(M) * q * s1
    ga = c1 + float(M * M) * q
    zero = jnp.zeros_like(al)
    mz = jnp.concatenate([al, be, ga, zero, zero, zero, zero, zero], axis=1)
    g2f = jnp.dot(mz, r_ref[...], preferred_element_type=jnp.float32) + bfc_ref[...]
    pf = g2f[:, :4]
    pc = g2f[:, 4:]
    out_ref[...] = _softplus(_sigmoid(pf) * pc)


def kernel(atom_fea, nbr_fea, state_fea, atom_nbr_idx, node_atom_idx,
           W_atom, b_atom, W_embed, b_embed, W_fc, b_fc):
    N = atom_fea.shape[0]
    n_graph = state_fea.shape[0]
    nblk = N // BN

    W_self = W_atom[:A]
    W_nbr = W_atom[A:2 * A]
    W_e = W_atom[2 * A:2 * A + NB]
    W_st = W_atom[2 * A + NB:]
    eye_m = jnp.eye(M, dtype=jnp.float32)
    bdn = jnp.concatenate([jnp.kron(eye_m, W_nbr[:, :A]),
                           jnp.kron(eye_m, W_nbr[:, A:])], axis=1)
    bde = jnp.concatenate([jnp.kron(eye_m, W_e[:, :A]),
                           jnp.kron(eye_m, W_e[:, A:])], axis=1)
    tile_i = np.tile(np.eye(A, dtype=np.float32), (1, M))
    t1m = np.block([[tile_i, np.zeros_like(tile_i)],
                    [np.zeros_like(tile_i), tile_i]])
    s16 = np.tile(np.eye(A, dtype=np.float32), (M, 1))
    s4 = np.tile(np.eye(NB, dtype=np.float32), (M, 1))
    ssm = np.zeros((M * NB, M), dtype=np.float32)
    ssm[np.arange(M) * NB, np.arange(M)] = 1.0
    xsm = np.kron(np.eye(M, dtype=np.float32), np.ones((1, NB), np.float32))
    x16 = np.kron(np.eye(M, dtype=np.float32), np.ones((1, A), np.float32))
    ba2 = b_atom.reshape(1, 2 * A)
    state_pad = jnp.zeros((NGP, state_fea.shape[1]), jnp.float32).at[:n_graph].set(state_fea)
    w_emb = W_embed.sum(axis=0)
    wr = W_fc.reshape(A, 8, 8)
    r_a = jnp.einsum('i,k,iko->o', w_emb, w_emb[:8], wr)
    r_b = (jnp.einsum('i,k,iko->o', w_emb, b_embed[:8], wr)
           + jnp.einsum('i,k,iko->o', b_embed, w_emb[:8], wr))
    r_g = jnp.einsum('i,k,iko->o', b_embed, b_embed[:8], wr)
    rmat = jnp.zeros((8, 8), jnp.float32).at[0].set(r_a).at[1].set(r_b).at[2].set(r_g)
    bfc2 = b_fc.reshape(1, 8)

    idx2 = atom_nbr_idx.astype(jnp.int32).reshape(N * M)
    ni2 = node_atom_idx.astype(jnp.int32).reshape(N, 1)
    nbrrow = nbr_fea.reshape(N, M * NB)

    g1row = _sc_gather(atom_fea, idx2).reshape(N, M * A)

    blk = lambda shape: pl.BlockSpec(shape, lambda i: (i, 0))
    full = lambda shape: pl.BlockSpec(shape, lambda i: (0, 0))
    atom_out, ex = pl.pallas_call(
        _atom_body,
        grid=(nblk,),
        in_specs=[
            blk((BN, A)), blk((BN, M * A)), blk((BN, M * NB)), blk((BN, 1)),
            full((NGP, state_fea.shape[1])), full((A, 2 * A)),
            full((state_fea.shape[1], 2 * A)), full((1, 2 * A)),
            full((M * A, 2 * M * A)), full((M * NB, 2 * M * A)),
            full((2 * A, 2 * M * A)), full((M * A, A)), full((M * NB, NB)),
            full((M * NB, M)), full((M, M * NB)),
        ],
        out_specs=[blk((BN, A)), blk((BN, 2 * NB + M))],
        out_shape=[jax.ShapeDtypeStruct((N, A), jnp.float32),
                   jax.ShapeDtypeStruct((N, 2 * NB + M), jnp.float32)],
    )(atom_fea, g1row, nbrrow, ni2, state_pad, W_self, W_st, ba2, bdn, bde,
      jnp.asarray(t1m), jnp.asarray(s16), jnp.asarray(s4), jnp.asarray(ssm),
      jnp.asarray(xsm))

    g2row = _sc_gather(atom_out, idx2).reshape(N, M * A)

    acc = pl.pallas_call(
        _sym_body,
        grid=(nblk,),
        in_specs=[
            blk((BN, M * A)), blk((BN, 2 * NB + M)), blk((BN, 1)),
            full((M * A, A)), full((M, M * A)),
        ],
        out_specs=pl.BlockSpec((NGP, 8), lambda i: (0, 0)),
        out_shape=jax.ShapeDtypeStruct((NGP, 8), jnp.float32),
    )(g2row, ex, ni2, jnp.asarray(s16), jnp.asarray(x16))

    state_out = pl.pallas_call(
        _final_body,
        in_specs=[
            pl.BlockSpec((NGP, 8), lambda: (0, 0)),
            pl.BlockSpec((NGP, state_fea.shape[1]), lambda: (0, 0)),
            pl.BlockSpec((8, 8), lambda: (0, 0)),
            pl.BlockSpec((1, 8), lambda: (0, 0)),
        ],
        out_specs=pl.BlockSpec((NGP, 4), lambda: (0, 0)),
        out_shape=jax.ShapeDtypeStruct((NGP, 4), jnp.float32),
    )(acc, state_pad, rmat, bfc2)

    return atom_out, nbr_fea, state_out[:n_graph]


# speedup vs baseline: 7.2613x; 1.2025x over previous
import functools
import numpy as np
import jax
import jax.numpy as jnp
from jax import lax
from jax.experimental import pallas as pl
from jax.experimental.pallas import tpu as pltpu
from jax.experimental.pallas import tpu_sc as plsc

M = 16
A = 16
NB = 4
NGP = 512
BN = 2000
DMA_ROWS = 40
KJ = 25
NP = 102400


def _softplus(x):
    return jnp.maximum(x, 0.0) + jnp.log(1.0 + jnp.exp(-jnp.abs(x)))


def _sigmoid(x):
    return 1.0 / (1.0 + jnp.exp(-x))


def _sc_gather(table, idx_flat):
    B = idx_flat.shape[0]
    info = plsc.get_sparse_core_info()
    nc, ns = info.num_cores, info.num_subcores
    nw = nc * ns
    chunk = KJ * DMA_ROWS
    n_outer = B // (nw * chunk)

    mesh = plsc.VectorSubcoreMesh(core_axis_name="c", subcore_axis_name="s")

    @functools.partial(
        pl.kernel,
        mesh=mesh,
        compiler_params=pltpu.CompilerParams(use_tc_tiling_on_sc=False),
        out_type=jax.ShapeDtypeStruct((B, A), jnp.float32),
        scratch_types=[
            pltpu.VMEM((chunk,), jnp.int32),
            pltpu.VMEM((chunk,), jnp.int32),
            pltpu.VMEM((chunk, A), jnp.float32),
            pltpu.VMEM((chunk, A), jnp.float32),
            pltpu.SemaphoreType.DMA,
            pltpu.SemaphoreType.DMA,
            pltpu.SemaphoreType.DMA,
        ],
    )
    def gather_kernel(table_hbm, idx_hbm, out_hbm, idx_v0, idx_v1,
                      rows_v0, rows_v1, sem_i, sem_g, sem_w):
        wid = lax.axis_index("s") * nc + lax.axis_index("c")
        base = wid * n_outer
        idx_bufs = [idx_v0, idx_v1]
        row_bufs = [rows_v0, rows_v1]
        pltpu.async_copy(idx_hbm.at[pl.ds(base * chunk, chunk)], idx_v0, sem_i)

        def body(t, carry):
            for b in range(2):
                i = t * 2 + b
                g = base + i
                pltpu.make_async_copy(
                    idx_hbm.at[pl.ds(0, chunk)], idx_bufs[b], sem_i).wait()

                @pl.when(i + 1 < n_outer)
                def _():
                    pltpu.async_copy(
                        idx_hbm.at[pl.ds((g + 1) * chunk, chunk)],
                        idx_bufs[1 - b], sem_i)

                @pl.when(i >= 2)
                def _():
                    pltpu.make_async_copy(
                        out_hbm.at[pl.ds(0, chunk)], row_bufs[b], sem_w).wait()

                copies = [
                    pltpu.async_copy(
                        table_hbm.at[idx_bufs[b].at[pl.ds(j * DMA_ROWS, DMA_ROWS)]],
                        row_bufs[b].at[pl.ds(j * DMA_ROWS, DMA_ROWS)],
                        sem_g,
                    )
                    for j in range(KJ)
                ]
                for c in copies:
                    c.wait()
                pltpu.async_copy(row_bufs[b], out_hbm.at[pl.ds(g * chunk, chunk)],
                                 sem_w)
            return carry

        lax.fori_loop(0, n_outer // 2, body, 0)
        pltpu.make_async_copy(out_hbm.at[pl.ds(0, chunk)], rows_v0, sem_w).wait()
        pltpu.make_async_copy(out_hbm.at[pl.ds(0, chunk)], rows_v1, sem_w).wait()

    return gather_kernel(table, idx_flat)


def _sc_segsum(vals, idx, zeros):
    info = plsc.get_sparse_core_info()
    nc, ns = info.num_cores, info.num_subcores
    nw = nc * ns
    per_w = NP // nw
    ch = 800
    n_it = per_w // ch

    mesh = plsc.VectorSubcoreMesh(core_axis_name="c", subcore_axis_name="s")

    @functools.partial(
        pl.kernel,
        mesh=mesh,
        compiler_params=pltpu.CompilerParams(use_tc_tiling_on_sc=False),
        out_type=jax.ShapeDtypeStruct((2, NGP, A), jnp.float32),
        scratch_types=[
            pltpu.VMEM((ch, A), jnp.float32),
            pltpu.VMEM((ch,), jnp.int32),
            pltpu.VMEM_SHARED((NGP, A), jnp.float32),
        ],
    )
    def seg_kernel(vals_hbm, idx_hbm, zeros_hbm, out_hbm, v_v, i_v, acc_sh):
        cid = lax.axis_index("c")
        sid = lax.axis_index("s")
        wid = sid * nc + cid

        @pl.when(sid == 0)
        def _():
            pltpu.sync_copy(zeros_hbm, acc_sh)

        plsc.subcore_barrier()

        def body(i, carry):
            b0 = wid * per_w + i * ch
            pltpu.sync_copy(vals_hbm.at[pl.ds(b0, ch)], v_v)
            pltpu.sync_copy(idx_hbm.at[pl.ds(b0, ch)], i_v)
            pltpu.sync_copy(v_v, acc_sh.at[i_v], add=True)
            return carry

        lax.fori_loop(0, n_it, body, 0)
        plsc.subcore_barrier()

        @pl.when(sid == 0)
        def _():
            pltpu.sync_copy(acc_sh, out_hbm.at[cid])

    return seg_kernel(vals, idx, zeros)


def _atom_body(af_ref, g1_ref, nb_ref, ni_ref, sp_ref, wself_ref, wst_ref,
               ba_ref, bdn_ref, bde_ref, t1m_ref, s16_ref, s4_ref, ssm_ref,
               xsm_ref, out_ref, ex_ref):
    af = af_ref[...]
    g1 = g1_ref[...]
    nb = nb_ref[...]
    ni = ni_ref[...]
    oh = (lax.broadcasted_iota(jnp.int32, (BN, NGP), 1) == ni).astype(jnp.float32)
    st = jnp.dot(oh, sp_ref[...], preferred_element_type=jnp.float32)
    t1 = (jnp.dot(af, wself_ref[...], preferred_element_type=jnp.float32)
          + jnp.dot(st, wst_ref[...], preferred_element_type=jnp.float32)
          + ba_ref[...])
    fc = (jnp.dot(g1, bdn_ref[...], preferred_element_type=jnp.float32)
          + jnp.dot(nb, bde_ref[...], preferred_element_type=jnp.float32)
          + jnp.dot(t1, t1m_ref[...], preferred_element_type=jnp.float32))
    filt = fc[:, :M * A]
    core = fc[:, M * A:]
    h = _sigmoid(filt) * _softplus(core)
    nbr_sumed = jnp.dot(h, s16_ref[...], preferred_element_type=jnp.float32)
    out_ref[...] = _softplus(af + nbr_sumed)
    sm = jnp.dot(nb, ssm_ref[...], preferred_element_type=jnp.float32)
    sn = jnp.dot(nb, s4_ref[...], preferred_element_type=jnp.float32)
    smexp = jnp.dot(sm, xsm_ref[...], preferred_element_type=jnp.float32)
    sns = jnp.dot(smexp * nb, s4_ref[...], preferred_element_type=jnp.float32)
    ex_ref[...] = jnp.concatenate([sn, sns, sm], axis=1)


def _sym_body(g2_ref, ex_ref, s16_ref, x16_ref, acc_ref):
    g2 = g2_ref[...]
    ex = ex_ref[...]
    sn = ex[:, 0:NB]
    sns = ex[:, NB:2 * NB]
    sm = ex[:, 2 * NB:2 * NB + M]
    gsum = jnp.dot(g2, s16_ref[...], preferred_element_type=jnp.float32)
    smexp = jnp.dot(sm, x16_ref[...], preferred_element_type=jnp.float32)
    gwsum = jnp.dot(smexp * g2, s16_ref[...], preferred_element_type=jnp.float32)
    a1 = (jnp.sum(sns * sns, axis=1, keepdims=True)
          + jnp.sum(gwsum * gwsum, axis=1, keepdims=True))
    b1 = (jnp.sum(sns * sn, axis=1, keepdims=True)
          + jnp.sum(gwsum * gsum, axis=1, keepdims=True))
    c1 = (jnp.sum(sn * sn, axis=1, keepdims=True)
          + jnp.sum(gsum * gsum, axis=1, keepdims=True))
    s1 = jnp.sum(sm, axis=1, keepdims=True)
    ones = jnp.ones_like(s1)
    pad = jnp.zeros((BN, A - 6), jnp.float32)
    acc_ref[...] = jnp.concatenate([a1, b1, c1, s1, s1 * s1, ones, pad], axis=1)


def _final_body(seg_ref, sp_ref, r_ref, bfc_ref, out_ref):
    seg = seg_ref[...]
    acc = seg[:NGP] + seg[NGP:]
    cnt = jnp.maximum(acc[:, 5:6], 1.0)
    a1 = acc[:, 0:1] / cnt
    b1 = acc[:, 1:2] / cnt
    c1 = acc[:, 2:3] / cnt
    s1 = acc[:, 3:4] / cnt
    s2 = acc[:, 4:5] / cnt
    sp = sp_ref[...]
    q = jnp.sum(sp * sp, axis=1, keepdims=True)
    al = a1 + q * s2
    be = b1 + float(M) * q * s1
    ga = c1 + float(M * M) * q
    zero = jnp.zeros_like(al)
    mz = jnp.concatenate([al, be, ga, zero, zero, zero, zero, zero], axis=1)
    g2f = jnp.dot(mz, r_ref[...], preferred_element_type=jnp.float32) + bfc_ref[...]
    pf = g2f[:, :4]
    pc = g2f[:, 4:]
    out_ref[...] = _softplus(_sigmoid(pf) * pc)


def kernel(atom_fea, nbr_fea, state_fea, atom_nbr_idx, node_atom_idx,
           W_atom, b_atom, W_embed, b_embed, W_fc, b_fc):
    N = atom_fea.shape[0]
    n_graph = state_fea.shape[0]
    nblk = N // BN

    W_self = W_atom[:A]
    W_nbr = W_atom[A:2 * A]
    W_e = W_atom[2 * A:2 * A + NB]
    W_st = W_atom[2 * A + NB:]
    eye_m = jnp.eye(M, dtype=jnp.float32)
    bdn = jnp.concatenate([jnp.kron(eye_m, W_nbr[:, :A]),
                           jnp.kron(eye_m, W_nbr[:, A:])], axis=1)
    bde = jnp.concatenate([jnp.kron(eye_m, W_e[:, :A]),
                           jnp.kron(eye_m, W_e[:, A:])], axis=1)
    tile_i = np.tile(np.eye(A, dtype=np.float32), (1, M))
    t1m = np.block([[tile_i, np.zeros_like(tile_i)],
                    [np.zeros_like(tile_i), tile_i]])
    s16 = np.tile(np.eye(A, dtype=np.float32), (M, 1))
    s4 = np.tile(np.eye(NB, dtype=np.float32), (M, 1))
    ssm = np.zeros((M * NB, M), dtype=np.float32)
    ssm[np.arange(M) * NB, np.arange(M)] = 1.0
    xsm = np.kron(np.eye(M, dtype=np.float32), np.ones((1, NB), np.float32))
    x16 = np.kron(np.eye(M, dtype=np.float32), np.ones((1, A), np.float32))
    ba2 = b_atom.reshape(1, 2 * A)
    state_pad = jnp.zeros((NGP, state_fea.shape[1]), jnp.float32).at[:n_graph].set(state_fea)
    w_emb = W_embed.sum(axis=0)
    wr = W_fc.reshape(A, 8, 8)
    r_a = jnp.einsum('i,k,iko->o', w_emb, w_emb[:8], wr)
    r_b = (jnp.einsum('i,k,iko->o', w_emb, b_embed[:8], wr)
           + jnp.einsum('i,k,iko->o', b_embed, w_emb[:8], wr))
    r_g = jnp.einsum('i,k,iko->o', b_embed, b_embed[:8], wr)
    rmat = jnp.zeros((8, 8), jnp.float32).at[0].set(r_a).at[1].set(r_b).at[2].set(r_g)
    bfc2 = b_fc.reshape(1, 8)

    idx2 = atom_nbr_idx.astype(jnp.int32).reshape(N * M)
    ni2 = node_atom_idx.astype(jnp.int32).reshape(N, 1)
    nbrrow = nbr_fea.reshape(N, M * NB)

    g1row = _sc_gather(atom_fea, idx2).reshape(N, M * A)

    blk = lambda shape: pl.BlockSpec(shape, lambda i: (i, 0))
    full = lambda shape: pl.BlockSpec(shape, lambda i: (0, 0))
    atom_out, ex = pl.pallas_call(
        _atom_body,
        grid=(nblk,),
        in_specs=[
            blk((BN, A)), blk((BN, M * A)), blk((BN, M * NB)), blk((BN, 1)),
            full((NGP, state_fea.shape[1])), full((A, 2 * A)),
            full((state_fea.shape[1], 2 * A)), full((1, 2 * A)),
            full((M * A, 2 * M * A)), full((M * NB, 2 * M * A)),
            full((2 * A, 2 * M * A)), full((M * A, A)), full((M * NB, NB)),
            full((M * NB, M)), full((M, M * NB)),
        ],
        out_specs=[blk((BN, A)), blk((BN, 2 * NB + M))],
        out_shape=[jax.ShapeDtypeStruct((N, A), jnp.float32),
                   jax.ShapeDtypeStruct((N, 2 * NB + M), jnp.float32)],
    )(atom_fea, g1row, nbrrow, ni2, state_pad, W_self, W_st, ba2, bdn, bde,
      jnp.asarray(t1m), jnp.asarray(s16), jnp.asarray(s4), jnp.asarray(ssm),
      jnp.asarray(xsm))

    g2row = _sc_gather(atom_out, idx2).reshape(N, M * A)

    vals = pl.pallas_call(
        _sym_body,
        grid=(nblk,),
        in_specs=[
            blk((BN, M * A)), blk((BN, 2 * NB + M)),
            full((M * A, A)), full((M, M * A)),
        ],
        out_specs=blk((BN, A)),
        out_shape=jax.ShapeDtypeStruct((N, A), jnp.float32),
    )(g2row, ex, jnp.asarray(s16), jnp.asarray(x16))

    vals_p = jnp.pad(vals, ((0, NP - N), (0, 0)))
    idx_p = jnp.pad(node_atom_idx.astype(jnp.int32), (0, NP - N),
                    constant_values=NGP - 1)
    seg = _sc_segsum(vals_p, idx_p, jnp.zeros((NGP, A), jnp.float32))
    seg2 = seg.reshape(2 * NGP, A)

    state_out = pl.pallas_call(
        _final_body,
        in_specs=[
            pl.BlockSpec((2 * NGP, A), lambda: (0, 0)),
            pl.BlockSpec((NGP, state_fea.shape[1]), lambda: (0, 0)),
            pl.BlockSpec((8, 8), lambda: (0, 0)),
            pl.BlockSpec((1, 8), lambda: (0, 0)),
        ],
        out_specs=pl.BlockSpec((NGP, 4), lambda: (0, 0)),
        out_shape=jax.ShapeDtypeStruct((NGP, 4), jnp.float32),
    )(seg2, state_pad, rmat, bfc2)

    return atom_out, nbr_fea, state_out[:n_graph]
